# Initial kernel scaffold; baseline (speedup 1.0000x reference)
import jax
import jax.numpy as jnp
from jax.experimental import pallas as pl


def kernel(x, edge_index, edge_attr, node_type, neighbor_idx, obs_nongraph, type_embed, W_in, b_in, W_msg0, b_msg0, W_upd0, b_upd0, W_msg1, b_msg1, W_upd1, b_upd1, Ws1, bs1, Ws2, bs2, Ws3, bs3, Wm1, bm1, Wm2, bm2, Wm3, bm3, Wa, ba):
    raise NotImplementedError("write your pallas kernel here")


# speedup vs baseline: 2.9316x; 2.9316x over previous
import functools

import jax
import jax.numpy as jnp
from jax import lax
from jax.experimental import pallas as pl
from jax.experimental.pallas import tpu as pltpu
from jax.experimental.pallas import tpu_sc as plsc

F32 = jnp.float32

_N = 10000
_E = 320000
_D = 128
_DE = 16
_H = 128
_HS = 256
_B = 1024
_NT = 8
_NTE = 2
_MAXN = 15
_DNG = 32
_A = 10

_HIGH = jax.lax.Precision.HIGHEST


def _dot(a, b):
    return jnp.dot(a, b, preferred_element_type=F32, precision=_HIGH)


_BN = 1000
_BE = 8000


def _input_body(nt_ref, x_ref, wx_ref, temb_ref, wt_ref, b_ref, o_ref):
    tb = _dot(temb_ref[...], wt_ref[...]) + b_ref[...]
    nt = nt_ref[...]
    iota = lax.broadcasted_iota(jnp.int32, (_BN, _NT), 1)
    oh = (iota == nt).astype(F32)
    o_ref[...] = jnp.maximum(_dot(x_ref[...], wx_ref[...]) + _dot(oh, tb), 0.0)


def _tc_input(node_type2, x, wx, temb, wt, b):
    grid = (_N // _BN,)
    return pl.pallas_call(
        _input_body,
        grid=grid,
        in_specs=[
            pl.BlockSpec((_BN, 1), lambda i: (i, 0)),
            pl.BlockSpec((_BN, _D), lambda i: (i, 0)),
            pl.BlockSpec((_D, _H), lambda i: (0, 0)),
            pl.BlockSpec((_NT, _NTE), lambda i: (0, 0)),
            pl.BlockSpec((_NTE, _H), lambda i: (0, 0)),
            pl.BlockSpec((1, _H), lambda i: (0, 0)),
        ],
        out_specs=pl.BlockSpec((_BN, _H), lambda i: (i, 0)),
        out_shape=jax.ShapeDtypeStruct((_N, _H), F32),
    )(node_type2, x, wx, temb, wt, b)


def _proj_body(h_ref, wa_ref, wb_ref, oa_ref, ob_ref):
    h = h_ref[...]
    oa_ref[...] = _dot(h, wa_ref[...])
    ob_ref[...] = _dot(h, wb_ref[...])


def _tc_proj(h, wa, wb):
    grid = (_N // _BN,)
    return pl.pallas_call(
        _proj_body,
        grid=grid,
        in_specs=[
            pl.BlockSpec((_BN, _H), lambda i: (i, 0)),
            pl.BlockSpec((_H, _H), lambda i: (0, 0)),
            pl.BlockSpec((_H, _H), lambda i: (0, 0)),
        ],
        out_specs=[
            pl.BlockSpec((_BN, _H), lambda i: (i, 0)),
            pl.BlockSpec((_BN, _H), lambda i: (i, 0)),
        ],
        out_shape=[
            jax.ShapeDtypeStruct((_N, _H), F32),
            jax.ShapeDtypeStruct((_N, _H), F32),
        ],
    )(h, wa, wb)


def _eproj_body(ea_ref, we_ref, bm_ref, o_ref):
    o_ref[...] = _dot(ea_ref[...], we_ref[...]) + bm_ref[...]


def _tc_eproj(ea, we, bm):
    grid = (_E // _BE,)
    return pl.pallas_call(
        _eproj_body,
        grid=grid,
        in_specs=[
            pl.BlockSpec((_BE, _DE), lambda i: (i, 0)),
            pl.BlockSpec((_DE, _H), lambda i: (0, 0)),
            pl.BlockSpec((1, _H), lambda i: (0, 0)),
        ],
        out_specs=pl.BlockSpec((_BE, _H), lambda i: (i, 0)),
        out_shape=jax.ShapeDtypeStruct((_E, _H), F32),
    )(ea, we, bm)


def _upd_body(h_ref, a0_ref, a1_ref, wa_ref, wb_ref, b_ref, o_ref):
    agg = a0_ref[...] + a1_ref[...]
    o_ref[...] = jnp.maximum(
        _dot(h_ref[...], wa_ref[...]) + _dot(agg, wb_ref[...]) + b_ref[...], 0.0
    )


def _tc_update(h, a0, a1, wa, wb, b):
    grid = (_N // _BN,)
    return pl.pallas_call(
        _upd_body,
        grid=grid,
        in_specs=[
            pl.BlockSpec((_BN, _H), lambda i: (i, 0)),
            pl.BlockSpec((_BN, _H), lambda i: (i, 0)),
            pl.BlockSpec((_BN, _H), lambda i: (i, 0)),
            pl.BlockSpec((_H, _H), lambda i: (0, 0)),
            pl.BlockSpec((_H, _H), lambda i: (0, 0)),
            pl.BlockSpec((1, _H), lambda i: (0, 0)),
        ],
        out_specs=pl.BlockSpec((_BN, _H), lambda i: (i, 0)),
        out_shape=jax.ShapeDtypeStruct((_N, _H), F32),
    )(h, a0, a1, wa, wb, b)


def _scorer_body(h_ref, w1_ref, b1_ref, w2_ref, b2_ref, w3_ref, b3_ref, o_ref):
    t = jnp.maximum(_dot(h_ref[...], w1_ref[...]) + b1_ref[...], 0.0)
    t = jnp.maximum(_dot(t, w2_ref[...]) + b2_ref[...], 0.0)
    o_ref[...] = _dot(t, w3_ref[...]) + b3_ref[...]


def _tc_scorer(h, w1, b1, w2, b2, w3, b3):
    grid = (_N // _BN,)
    return pl.pallas_call(
        _scorer_body,
        grid=grid,
        in_specs=[
            pl.BlockSpec((_BN, _H), lambda i: (i, 0)),
            pl.BlockSpec((_H, _HS), lambda i: (0, 0)),
            pl.BlockSpec((1, _HS), lambda i: (0, 0)),
            pl.BlockSpec((_HS, _HS), lambda i: (0, 0)),
            pl.BlockSpec((1, _HS), lambda i: (0, 0)),
            pl.BlockSpec((_HS, 1), lambda i: (0, 0)),
            pl.BlockSpec((1, 1), lambda i: (0, 0)),
        ],
        out_specs=pl.BlockSpec((_BN, 1), lambda i: (i, 0)),
        out_shape=jax.ShapeDtypeStruct((_N, 1), F32),
    )(h, w1, b1, w2, b2, w3, b3)


def _head_body(sc_ref, obs_ref, w1a_ref, w1b_ref, b1_ref, w2_ref, b2_ref,
               w3_ref, b3_ref, wa_ref, ba_ref, o_ref):
    f = jnp.maximum(
        _dot(sc_ref[...], w1a_ref[...]) + _dot(obs_ref[...], w1b_ref[...])
        + b1_ref[...], 0.0)
    f = jnp.maximum(_dot(f, w2_ref[...]) + b2_ref[...], 0.0)
    f = jnp.maximum(_dot(f, w3_ref[...]) + b3_ref[...], 0.0)
    o_ref[...] = _dot(f, wa_ref[...]) + ba_ref[...]


def _tc_head(scores, obs, w1a, w1b, b1, w2, b2, w3, b3, wa, ba):
    return pl.pallas_call(
        _head_body,
        out_shape=jax.ShapeDtypeStruct((_B, _A), F32),
    )(scores, obs, w1a, w1b, b1, w2, b2, w3, b3, wa, ba)


_NC = 2
_NS = 16
_NW = _NC * _NS
_EPW = _E // _NW
_CH = 80
_NCH = _EPW // _CH
_NPAD = 10240
_RPS = _NPAD // _NS
_RCH = 128
_NRC = _RPS // _RCH

_sc_mesh = plsc.VectorSubcoreMesh(
    core_axis_name="c", subcore_axis_name="s", num_cores=_NC, num_subcores=_NS
)


def _edge_body(psrc, pdst, pe, src_hbm, dst_hbm, out,
               src_v, dst_v, gs_v, gd_v, m_v, buf_v, agg_sh, sem1, sem2):
    c = lax.axis_index("c")
    s = lax.axis_index("s")
    wid = s * _NC + c

    def _zrow(r, _):
        for j in range(_H // 16):
            buf_v[r, pl.ds(j * 16, 16)] = jnp.zeros((16,), F32)
        return 0
    lax.fori_loop(0, _RCH, _zrow, 0)
    for r in range(_NRC):
        pltpu.sync_copy(buf_v, agg_sh.at[pl.ds(s * _RPS + r * _RCH, _RCH)])
    plsc.subcore_barrier()

    def _chunk(t, _):
        base = wid * _EPW + t * _CH
        pltpu.sync_copy(src_hbm.at[pl.ds(base, _CH)], src_v)
        pltpu.sync_copy(dst_hbm.at[pl.ds(base, _CH)], dst_v)
        cp1 = pltpu.async_copy(psrc.at[src_v], gs_v, sem1)
        cp2 = pltpu.async_copy(pdst.at[dst_v], gd_v, sem2)
        pltpu.sync_copy(pe.at[pl.ds(base, _CH)], m_v)
        cp1.wait()
        cp2.wait()

        def _erow(e, _):
            for j in range(_H // 16):
                sl = pl.ds(j * 16, 16)
                v = gs_v[e, sl] + gd_v[e, sl] + m_v[e, sl]
                m_v[e, sl] = jnp.maximum(v, 0.0)
            return 0
        lax.fori_loop(0, _CH, _erow, 0)

        pltpu.sync_copy(m_v, agg_sh.at[dst_v], add=True)
        return 0
    lax.fori_loop(0, _NCH, _chunk, 0)
    plsc.subcore_barrier()

    for r in range(_NRC):
        row = s * _RPS + r * _RCH
        pltpu.sync_copy(agg_sh.at[pl.ds(row, _RCH)], buf_v)
        pltpu.sync_copy(buf_v, out.at[c, pl.ds(row, _RCH)])


def _sc_edge(psrc, pdst, pe, src, dst):
    k = pl.kernel(
        _edge_body,
        out_type=jax.ShapeDtypeStruct((_NC, _NPAD, _H), F32),
        mesh=_sc_mesh,
        scratch_types=[
            pltpu.VMEM((_CH,), jnp.int32),
            pltpu.VMEM((_CH,), jnp.int32),
            pltpu.VMEM((_CH, _H), F32),
            pltpu.VMEM((_CH, _H), F32),
            pltpu.VMEM((_CH, _H), F32),
            pltpu.VMEM((_RCH, _H), F32),
            pltpu.VMEM_SHARED((_NPAD, _H), F32),
            pltpu.SemaphoreType.DMA,
            pltpu.SemaphoreType.DMA,
        ],
    )
    return k(psrc, pdst, pe, src, dst)


_GPW = _B * _MAXN // _NW
_GC = 96


def _gather_body(s_hbm, idx_hbm, out_hbm, idx_v, o_v, sem):
    c = lax.axis_index("c")
    s = lax.axis_index("s")
    wid = s * _NC + c
    pltpu.sync_copy(idx_hbm.at[pl.ds(wid * _GPW, _GPW)], idx_v)
    for k in range(_GPW // _GC):
        sl = pl.ds(k * _GC, _GC)
        pltpu.async_copy(s_hbm.at[idx_v.at[sl]], o_v.at[sl], sem).wait()
    pltpu.sync_copy(o_v, out_hbm.at[pl.ds(wid * _GPW, _GPW)])


def _sc_gather(s_flat, idx_flat):
    k = pl.kernel(
        _gather_body,
        out_type=jax.ShapeDtypeStruct((_B * _MAXN,), F32),
        mesh=_sc_mesh,
        scratch_types=[
            pltpu.VMEM((_GPW,), jnp.int32),
            pltpu.VMEM((_GPW,), F32),
            pltpu.SemaphoreType.DMA,
        ],
    )
    return k(s_flat, idx_flat)


def kernel(x, edge_index, edge_attr, node_type, neighbor_idx, obs_nongraph,
           type_embed, W_in, b_in, W_msg0, b_msg0, W_upd0, b_upd0,
           W_msg1, b_msg1, W_upd1, b_upd1, Ws1, bs1, Ws2, bs2, Ws3, bs3,
           Wm1, bm1, Wm2, bm2, Wm3, bm3, Wa, ba):
    src = edge_index[0]
    dst = edge_index[1]
    nt2 = node_type.reshape(_N, 1)

    h = _tc_input(nt2, x, W_in[:_D], type_embed, W_in[_D:],
                  b_in.reshape(1, _H))

    for Wm, bm, Wu, bu in ((W_msg0, b_msg0, W_upd0, b_upd0),
                           (W_msg1, b_msg1, W_upd1, b_upd1)):
        psrc, pdst = _tc_proj(h, Wm[:_H], Wm[_H:2 * _H])
        pe = _tc_eproj(edge_attr, Wm[2 * _H:], bm.reshape(1, _H))
        agg2 = _sc_edge(psrc, pdst, pe, src, dst)
        h = _tc_update(h, agg2[0], agg2[1], Wu[:_H], Wu[_H:],
                       bu.reshape(1, _H))

    s = _tc_scorer(h, Ws1, bs1.reshape(1, _HS), Ws2, bs2.reshape(1, _HS),
                   Ws3, bs3.reshape(1, 1))

    scores = _sc_gather(s.reshape(_N), neighbor_idx.reshape(_B * _MAXN))

    logits = _tc_head(scores.reshape(_B, _MAXN), obs_nongraph,
                      Wm1[:_MAXN], Wm1[_MAXN:], bm1.reshape(1, _HS),
                      Wm2, bm2.reshape(1, _HS), Wm3, bm3.reshape(1, _HS),
                      Wa, ba.reshape(1, _A))
    return logits


# speedup vs baseline: 4.1146x; 1.4035x over previous
import functools

import jax
import jax.numpy as jnp
from jax import lax
from jax.experimental import pallas as pl
from jax.experimental.pallas import tpu as pltpu
from jax.experimental.pallas import tpu_sc as plsc

F32 = jnp.float32

_N = 10000
_E = 320000
_D = 128
_DE = 16
_H = 128
_HS = 256
_B = 1024
_NT = 8
_NTE = 2
_MAXN = 15
_DNG = 32
_A = 10

_HIGH = jax.lax.Precision.HIGHEST


def _bfr(a):
    return a.astype(jnp.bfloat16).astype(F32)


def _dot(a, b):
    return jnp.dot(_bfr(a), _bfr(b), preferred_element_type=F32,
                   precision=_HIGH)


def _dot_exact(a, b):
    return jnp.dot(a, b, preferred_element_type=F32, precision=_HIGH)


_BN = 1000
_BE = 8000


def _input_body(nt_ref, x_ref, wx_ref, temb_ref, wt_ref, b_ref, o_ref):
    tb = _dot(temb_ref[...], wt_ref[...]) + b_ref[...]
    nt = nt_ref[...]
    iota = lax.broadcasted_iota(jnp.int32, (_BN, _NT), 1)
    oh = (iota == nt).astype(F32)
    o_ref[...] = jnp.maximum(
        _dot(x_ref[...], wx_ref[...]) + _dot_exact(oh, tb), 0.0)


def _tc_input(node_type2, x, wx, temb, wt, b):
    grid = (_N // _BN,)
    return pl.pallas_call(
        _input_body,
        grid=grid,
        in_specs=[
            pl.BlockSpec((_BN, 1), lambda i: (i, 0)),
            pl.BlockSpec((_BN, _D), lambda i: (i, 0)),
            pl.BlockSpec((_D, _H), lambda i: (0, 0)),
            pl.BlockSpec((_NT, _NTE), lambda i: (0, 0)),
            pl.BlockSpec((_NTE, _H), lambda i: (0, 0)),
            pl.BlockSpec((1, _H), lambda i: (0, 0)),
        ],
        out_specs=pl.BlockSpec((_BN, _H), lambda i: (i, 0)),
        out_shape=jax.ShapeDtypeStruct((_N, _H), F32),
    )(node_type2, x, wx, temb, wt, b)


def _proj_body(h_ref, wa_ref, wb_ref, oa_ref, ob_ref):
    h = h_ref[...]
    oa_ref[...] = _dot(h, wa_ref[...])
    ob_ref[...] = _dot(h, wb_ref[...])


def _tc_proj(h, wa, wb):
    grid = (_N // _BN,)
    return pl.pallas_call(
        _proj_body,
        grid=grid,
        in_specs=[
            pl.BlockSpec((_BN, _H), lambda i: (i, 0)),
            pl.BlockSpec((_H, _H), lambda i: (0, 0)),
            pl.BlockSpec((_H, _H), lambda i: (0, 0)),
        ],
        out_specs=[
            pl.BlockSpec((_BN, _H), lambda i: (i, 0)),
            pl.BlockSpec((_BN, _H), lambda i: (i, 0)),
        ],
        out_shape=[
            jax.ShapeDtypeStruct((_N, _H), F32),
            jax.ShapeDtypeStruct((_N, _H), F32),
        ],
    )(h, wa, wb)


def _eproj_body(ea_ref, we_ref, bm_ref, o_ref):
    o_ref[...] = _dot(ea_ref[...], we_ref[...]) + bm_ref[...]


def _tc_eproj(ea, we, bm):
    grid = (_E // _BE,)
    return pl.pallas_call(
        _eproj_body,
        grid=grid,
        in_specs=[
            pl.BlockSpec((_BE, _DE), lambda i: (i, 0)),
            pl.BlockSpec((_DE, _H), lambda i: (0, 0)),
            pl.BlockSpec((1, _H), lambda i: (0, 0)),
        ],
        out_specs=pl.BlockSpec((_BE, _H), lambda i: (i, 0)),
        out_shape=jax.ShapeDtypeStruct((_E, _H), F32),
    )(ea, we, bm)


def _upd_body(h_ref, a0_ref, a1_ref, wa_ref, wb_ref, b_ref, o_ref):
    agg = a0_ref[...] + a1_ref[...]
    o_ref[...] = jnp.maximum(
        _dot(h_ref[...], wa_ref[...]) + _dot(agg, wb_ref[...]) + b_ref[...], 0.0
    )


def _tc_update(h, a0, a1, wa, wb, b):
    grid = (_N // _BN,)
    return pl.pallas_call(
        _upd_body,
        grid=grid,
        in_specs=[
            pl.BlockSpec((_BN, _H), lambda i: (i, 0)),
            pl.BlockSpec((_BN, _H), lambda i: (i, 0)),
            pl.BlockSpec((_BN, _H), lambda i: (i, 0)),
            pl.BlockSpec((_H, _H), lambda i: (0, 0)),
            pl.BlockSpec((_H, _H), lambda i: (0, 0)),
            pl.BlockSpec((1, _H), lambda i: (0, 0)),
        ],
        out_specs=pl.BlockSpec((_BN, _H), lambda i: (i, 0)),
        out_shape=jax.ShapeDtypeStruct((_N, _H), F32),
    )(h, a0, a1, wa, wb, b)


def _scorer_body(h_ref, w1_ref, b1_ref, w2_ref, b2_ref, w3_ref, b3_ref, o_ref):
    t = jnp.maximum(_dot(h_ref[...], w1_ref[...]) + b1_ref[...], 0.0)
    t = jnp.maximum(_dot(t, w2_ref[...]) + b2_ref[...], 0.0)
    o_ref[...] = _dot(t, w3_ref[...]) + b3_ref[...]


def _tc_scorer(h, w1, b1, w2, b2, w3, b3):
    grid = (_N // _BN,)
    return pl.pallas_call(
        _scorer_body,
        grid=grid,
        in_specs=[
            pl.BlockSpec((_BN, _H), lambda i: (i, 0)),
            pl.BlockSpec((_H, _HS), lambda i: (0, 0)),
            pl.BlockSpec((1, _HS), lambda i: (0, 0)),
            pl.BlockSpec((_HS, _HS), lambda i: (0, 0)),
            pl.BlockSpec((1, _HS), lambda i: (0, 0)),
            pl.BlockSpec((_HS, 1), lambda i: (0, 0)),
            pl.BlockSpec((1, 1), lambda i: (0, 0)),
        ],
        out_specs=pl.BlockSpec((_BN, 1), lambda i: (i, 0)),
        out_shape=jax.ShapeDtypeStruct((_N, 1), F32),
    )(h, w1, b1, w2, b2, w3, b3)


def _head_body(sc_ref, obs_ref, w1a_ref, w1b_ref, b1_ref, w2_ref, b2_ref,
               w3_ref, b3_ref, wa_ref, ba_ref, o_ref):
    f = jnp.maximum(
        _dot(sc_ref[...], w1a_ref[...]) + _dot(obs_ref[...], w1b_ref[...])
        + b1_ref[...], 0.0)
    f = jnp.maximum(_dot(f, w2_ref[...]) + b2_ref[...], 0.0)
    f = jnp.maximum(_dot(f, w3_ref[...]) + b3_ref[...], 0.0)
    o_ref[...] = _dot(f, wa_ref[...]) + ba_ref[...]


def _tc_head(scores, obs, w1a, w1b, b1, w2, b2, w3, b3, wa, ba):
    return pl.pallas_call(
        _head_body,
        out_shape=jax.ShapeDtypeStruct((_B, _A), F32),
    )(scores, obs, w1a, w1b, b1, w2, b2, w3, b3, wa, ba)


_NC = 2
_NS = 16
_NW = _NC * _NS
_EPW = _E // _NW
_CH = 40
_NCH = _EPW // _CH
_RCH = _CH
_NRC = _N // _RCH

_sc_mesh = plsc.VectorSubcoreMesh(
    core_axis_name="c", subcore_axis_name="s", num_cores=_NC, num_subcores=_NS
)


def _edge_body(psrc, pdst, pe, idx3, out,
               idx_v, gs0, gd0, m0, gs1, gd1, m1,
               agg_sh, sem0, sem1, isem):
    c = lax.axis_index("c")
    s = lax.axis_index("s")
    wid = s * _NC + c

    data = ((gs0, gd0, m0, sem0), (gs1, gd1, m1, sem1))

    def _zrow(r, _):
        for j in range(_H // 16):
            m0[r, pl.ds(j * 16, 16)] = jnp.zeros((16,), F32)
        return 0
    lax.fori_loop(0, _RCH, _zrow, 0)
    for k in range(16):
        cid = s + _NS * k

        @pl.when(cid < _NRC)
        def _():
            pltpu.sync_copy(m0, agg_sh.at[pl.ds(cid * _RCH, _RCH)])
    plsc.subcore_barrier()

    def _issue_idx(t, slot):
        pltpu.async_copy(idx3.at[wid, t], idx_v.at[slot], isem)

    def _drain_idx():
        pltpu.make_async_copy(idx3.at[wid, 0], idx_v.at[0], isem).wait()

    def _issue_g(t, slot, p):
        gs, gd, m, sem = data[p]
        pltpu.async_copy(psrc.at[idx_v.at[slot, 0]], gs, sem)
        pltpu.async_copy(pdst.at[idx_v.at[slot, 1]], gd, sem)
        pltpu.async_copy(pe.at[pl.ds(wid * _EPW + t * _CH, _CH)], m, sem)

    def _drain_g(p):
        gs, gd, m, sem = data[p]
        pltpu.make_async_copy(pe.at[pl.ds(0, _CH)], gs, sem).wait()
        pltpu.make_async_copy(pe.at[pl.ds(0, _CH)], gd, sem).wait()
        pltpu.make_async_copy(pe.at[pl.ds(0, _CH)], m, sem).wait()

    def _compute(p):
        gs, gd, m, _ = data[p]

        def _erow(e, _):
            for j in range(_H // 16):
                sl = pl.ds(j * 16, 16)
                v = gs[e, sl] + gd[e, sl] + m[e, sl]
                m[e, sl] = jnp.maximum(v, 0.0)
            return 0
        lax.fori_loop(0, _CH, _erow, 0)

    def _halfstep(i, p):
        t = 2 * i + p
        r = lax.rem(t, 3)
        r2 = lax.rem(t + 2, 3)
        _drain_g(p)
        _compute(p)
        pltpu.sync_copy(data[p][2], agg_sh.at[idx_v.at[r, 1]], add=True)

        @pl.when(t + 2 < _NCH)
        def _():
            _drain_idx()

        @pl.when(t + 3 < _NCH)
        def _():
            _issue_idx(t + 3, r)

        @pl.when(t + 2 < _NCH)
        def _():
            _issue_g(t + 2, r2, p)

    _issue_idx(0, 0)
    _drain_idx()
    _issue_idx(1, 1)
    _drain_idx()
    _issue_g(0, 0, 0)
    _issue_g(1, 1, 1)
    _issue_idx(2, 2)

    def _pair(i, _):
        _halfstep(i, 0)
        _halfstep(i, 1)
        return 0
    lax.fori_loop(0, _NCH // 2, _pair, 0)
    plsc.subcore_barrier()

    for k in range(16):
        cid = s + _NS * k

        @pl.when(cid < _NRC)
        def _():
            pltpu.sync_copy(agg_sh.at[pl.ds(cid * _RCH, _RCH)], m0)
            pltpu.sync_copy(m0, out.at[c, pl.ds(cid * _RCH, _RCH)])


def _sc_edge(psrc, pdst, pe, idx3):
    k = pl.kernel(
        _edge_body,
        out_type=jax.ShapeDtypeStruct((_NC, _N, _H), F32),
        mesh=_sc_mesh,
        scratch_types=[
            pltpu.VMEM((3, 2, _CH), jnp.int32),
            pltpu.VMEM((_CH, _H), F32),
            pltpu.VMEM((_CH, _H), F32),
            pltpu.VMEM((_CH, _H), F32),
            pltpu.VMEM((_CH, _H), F32),
            pltpu.VMEM((_CH, _H), F32),
            pltpu.VMEM((_CH, _H), F32),
            pltpu.VMEM_SHARED((_N, _H), F32),
            pltpu.SemaphoreType.DMA,
            pltpu.SemaphoreType.DMA,
            pltpu.SemaphoreType.DMA,
        ],
    )
    return k(psrc, pdst, pe, idx3)


_GPW = _B * _MAXN // _NW
_GC = 96


def _gather_body(s_hbm, idx_hbm, out_hbm, idx_v, o_v, sem):
    c = lax.axis_index("c")
    s = lax.axis_index("s")
    wid = s * _NC + c
    pltpu.sync_copy(idx_hbm.at[pl.ds(wid * _GPW, _GPW)], idx_v)
    for k in range(_GPW // _GC):
        sl = pl.ds(k * _GC, _GC)
        pltpu.async_copy(s_hbm.at[idx_v.at[sl]], o_v.at[sl], sem).wait()
    pltpu.sync_copy(o_v, out_hbm.at[pl.ds(wid * _GPW, _GPW)])


def _sc_gather(s_flat, idx_flat):
    k = pl.kernel(
        _gather_body,
        out_type=jax.ShapeDtypeStruct((_B * _MAXN,), F32),
        mesh=_sc_mesh,
        scratch_types=[
            pltpu.VMEM((_GPW,), jnp.int32),
            pltpu.VMEM((_GPW,), F32),
            pltpu.SemaphoreType.DMA,
        ],
    )
    return k(s_flat, idx_flat)


def kernel(x, edge_index, edge_attr, node_type, neighbor_idx, obs_nongraph,
           type_embed, W_in, b_in, W_msg0, b_msg0, W_upd0, b_upd0,
           W_msg1, b_msg1, W_upd1, b_upd1, Ws1, bs1, Ws2, bs2, Ws3, bs3,
           Wm1, bm1, Wm2, bm2, Wm3, bm3, Wa, ba):
    idx3 = jnp.stack(
        [edge_index[0].reshape(_NW, _NCH, _CH),
         edge_index[1].reshape(_NW, _NCH, _CH)], axis=2)
    nt2 = node_type.reshape(_N, 1)

    h = _tc_input(nt2, x, W_in[:_D], type_embed, W_in[_D:],
                  b_in.reshape(1, _H))

    for Wm, bm, Wu, bu in ((W_msg0, b_msg0, W_upd0, b_upd0),
                           (W_msg1, b_msg1, W_upd1, b_upd1)):
        psrc, pdst = _tc_proj(h, Wm[:_H], Wm[_H:2 * _H])
        pe = _tc_eproj(edge_attr, Wm[2 * _H:], bm.reshape(1, _H))
        agg2 = _sc_edge(psrc, pdst, pe, idx3)
        h = _tc_update(h, agg2[0], agg2[1], Wu[:_H], Wu[_H:],
                       bu.reshape(1, _H))

    s = _tc_scorer(h, Ws1, bs1.reshape(1, _HS), Ws2, bs2.reshape(1, _HS),
                   Ws3, bs3.reshape(1, 1))

    scores = _sc_gather(s.reshape(_N), neighbor_idx.reshape(_B * _MAXN))

    logits = _tc_head(scores.reshape(_B, _MAXN), obs_nongraph,
                      Wm1[:_MAXN], Wm1[_MAXN:], bm1.reshape(1, _HS),
                      Wm2, bm2.reshape(1, _HS), Wm3, bm3.reshape(1, _HS),
                      Wa, ba.reshape(1, _A))
    return logits
